# baseline (speedup 1.0000x reference)
import jax
import jax.numpy as jnp
from jax.experimental import pallas as pl

_B = 512
_BBLK = 128
_N = 68
_NP = 72
_K = 15
_D = 128
_F = _K * _D
_CA = 8


def _cos_classifier_body(emb_ref, pw_ref, xa_ref, pa_ref, out_ref):
    x = emb_ref[:, :_F]
    p = pw_ref[:, :_F]
    ones_f = jnp.ones((1, _F), dtype=jnp.float32)

    xn2 = jax.lax.dot_general(ones_f, x * x, (((1,), (1,)), ((), ())),
                              preferred_element_type=jnp.float32)
    pn2 = jax.lax.dot_general(p * p, ones_f, (((1,), (1,)), ((), ())),
                              preferred_element_type=jnp.float32)
    rx = jax.lax.rsqrt(jnp.maximum(xn2, 1e-24))
    rp = jax.lax.rsqrt(jnp.maximum(pn2, 1e-24))

    d2 = jax.lax.dot_general(pa_ref[...], xa_ref[...],
                             (((1,), (1,)), ((), ())),
                             preferred_element_type=jnp.float32)
    t = jnp.sqrt(jnp.maximum(d2.reshape(_K, _NP, _BBLK), 0.0))
    m = jnp.max(t, axis=0, keepdims=True)
    e = jnp.exp(t - m)
    s = jnp.sum(e, axis=0, keepdims=True)
    w2 = e * ((240.0 * rx[None]) / s)

    acc = jnp.zeros((_N, _BBLK), dtype=jnp.float32)
    for k in range(_K):
        sk = jax.lax.dot_general(
            p[:, k * _D:(k + 1) * _D], x[:, k * _D:(k + 1) * _D],
            dimension_numbers=(((1,), (1,)), ((), ())),
            preferred_element_type=jnp.float32,
        )
        acc = acc + w2[k, :_N, :] * sk
    out_ref[...] = jnp.transpose(acc * rp)


@jax.jit
def kernel(emb, proto_w):
    u = emb[:, _F:].reshape(_B, _K, 3) * (1.0 / 200.0)
    v = proto_w[:, _F:].reshape(_N, _K, 3) * (1.0 / 200.0)
    xa = jnp.concatenate(
        [u, jnp.sum(u * u, axis=2, keepdims=True),
         jnp.ones((_B, _K, 1), jnp.float32),
         jnp.zeros((_B, _K, _CA - 5), jnp.float32)], axis=2)
    xa = xa.reshape(_B, _K * _CA)
    pv = jnp.concatenate(
        [-2.0 * v, jnp.ones((_N, _K, 1), jnp.float32),
         jnp.sum(v * v, axis=2, keepdims=True),
         jnp.zeros((_N, _K, _CA - 5), jnp.float32)], axis=2)
    pv = jnp.pad(pv, ((0, _NP - _N), (0, 0), (0, 0)))
    pa = (jnp.transpose(pv, (1, 0, 2))[:, :, None, :]
          * jnp.eye(_K, dtype=jnp.float32)[:, None, :, None])
    pa = pa.reshape(_K * _NP, _K * _CA)
    grid = (_B // _BBLK,)
    return pl.pallas_call(
        _cos_classifier_body,
        grid=grid,
        in_specs=[
            pl.BlockSpec((_BBLK, emb.shape[1]), lambda i: (i, 0)),
            pl.BlockSpec((_N, proto_w.shape[1]), lambda i: (0, 0)),
            pl.BlockSpec((_BBLK, _K * _CA), lambda i: (i, 0)),
            pl.BlockSpec((_K * _NP, _K * _CA), lambda i: (0, 0)),
        ],
        out_specs=pl.BlockSpec((_BBLK, _N), lambda i: (i, 0)),
        out_shape=jax.ShapeDtypeStruct((_B, _N), jnp.float32),
    )(emb, proto_w, xa, pa)
